# baseline (speedup 1.0000x reference)
import functools

import jax
import jax.numpy as jnp
from jax import lax
from jax.experimental import pallas as pl
from jax.experimental.pallas import tpu as pltpu
from jax.experimental.pallas import tpu_sc as plsc

N = 10000
E = 320000
G = 64
D_IN = 128
H = 32

NC = 2
NS = 16
TILES = NC * NS
CHW = 128
PER = E // TILES
CH = 80
PERP = CH * CHW
NCHUNK = TILES * CH
CH0 = 56
CH1 = 104
NP = N + 112
ROWS = NP // NS

_mesh = plsc.VectorSubcoreMesh(core_axis_name="c", subcore_axis_name="s")


@functools.partial(
    pl.kernel,
    mesh=_mesh,
    out_type=jax.ShapeDtypeStruct((TILES, NP), jnp.float32),
    scratch_types=[
        pltpu.VMEM((PERP,), jnp.int32),
        pltpu.VMEM((NP,), jnp.float32),
    ],
    compiler_params=pltpu.CompilerParams(needs_layout_passes=False,
                                         use_tc_tiling_on_sc=False),
)
def _deg_kernel(dst_hbm, out_hbm, idx_v, deg_v):
    c = lax.axis_index("c")
    s = lax.axis_index("s")
    w = s * NC + c
    pltpu.sync_copy(dst_hbm.at[w], idx_v)
    zero16 = jnp.zeros((16,), jnp.float32)
    one16 = jnp.ones((16,), jnp.float32)

    def zb(i, carry):
        deg_v[pl.ds(i * 16, 16)] = zero16
        return carry

    lax.fori_loop(0, NP // 16, zb, 0, unroll=8)

    def sb(i, carry):
        idx = idx_v[pl.ds(i * 16, 16)]
        plsc.addupdate_scatter(deg_v, [idx], one16)
        return carry

    lax.fori_loop(0, PERP // 16, sb, 0, unroll=8)
    pltpu.sync_copy(deg_v, out_hbm.at[w])


@functools.partial(
    pl.kernel,
    mesh=_mesh,
    out_type=jax.ShapeDtypeStruct((NC, NP, H), jnp.float32),
    scratch_types=[
        pltpu.VMEM((CH1, CHW), jnp.int32),
        pltpu.VMEM((CH1, CHW), jnp.int32),
        pltpu.VMEM((CHW, H), jnp.float32),
        pltpu.VMEM((CHW, H), jnp.float32),
        pltpu.VMEM((CHW, H), jnp.float32),
        pltpu.VMEM((CHW, H), jnp.float32),
        pltpu.VMEM_SHARED((NP, H), jnp.float32),
        pltpu.SemaphoreType.DMA,
        pltpu.SemaphoreType.DMA,
        pltpu.SemaphoreType.DMA,
        pltpu.SemaphoreType.DMA,
    ],
    compiler_params=pltpu.CompilerParams(needs_layout_passes=False,
                                         use_tc_tiling_on_sc=False),
)
def _scatter_kernel(hp_hbm, src_hbm, dst_hbm, zeros_hbm, out_hbm,
                    src_v, dst_v, buf_a, buf_b, buf_c, buf_d, s_sh,
                    sem_a, sem_b, sem_c, sem_d):
    c = lax.axis_index("c")
    s = lax.axis_index("s")
    base = jnp.where(c == 0, s * CH0, NS * CH0 + s * CH1)
    pltpu.sync_copy(src_hbm.at[pl.ds(base, CH1)], src_v)
    pltpu.sync_copy(dst_hbm.at[pl.ds(base, CH1)], dst_v)
    pltpu.sync_copy(zeros_hbm.at[pl.ds(s * ROWS, ROWS)],
                    s_sh.at[pl.ds(s * ROWS, ROWS)])
    plsc.subcore_barrier()

    bufs = (buf_a, buf_b, buf_c, buf_d)
    sems = (sem_a, sem_b, sem_c, sem_d)
    DEPTH = 4

    def ring(nch):
        for p in range(DEPTH - 1):
            pltpu.async_copy(hp_hbm.at[src_v.at[p]], bufs[p], sems[p])

        def body(j, carry):
            slot = j % DEPTH
            nxt = j + DEPTH - 1
            for p in range(DEPTH):
                @pl.when(jnp.logical_and(nxt < nch, slot == p))
                def _(p=p):
                    pltpu.async_copy(hp_hbm.at[src_v.at[nxt]],
                                     bufs[(p + DEPTH - 1) % DEPTH],
                                     sems[(p + DEPTH - 1) % DEPTH])
            for p in range(DEPTH):
                @pl.when(slot == p)
                def _(p=p):
                    pltpu.make_async_copy(hp_hbm.at[src_v.at[j]],
                                          bufs[p], sems[p]).wait()
                    pltpu.sync_copy(bufs[p], s_sh.at[dst_v.at[j]], add=True)
            return carry

        lax.fori_loop(0, nch, body, 0)

    @pl.when(c == 0)
    def _():
        ring(CH0)

    @pl.when(c == 1)
    def _():
        ring(CH1)

    plsc.subcore_barrier()
    pltpu.sync_copy(s_sh.at[pl.ds(s * ROWS, ROWS)],
                    out_hbm.at[c, pl.ds(s * ROWS, ROWS)])


def _matmul_body(x_ref, w1_ref, h_ref):
    h_ref[...] = jnp.dot(x_ref[...], w1_ref[...],
                         preferred_element_type=jnp.float32)


def _scale_body(h_ref, degp_ref, hp_ref, dinv_ref):
    deg = jnp.sum(degp_ref[...], axis=0)[:N] + 1.0
    dinv = lax.rsqrt(deg)
    hp_ref[...] = h_ref[...] * dinv[:, None]
    dinv_ref[...] = dinv[:, None]


def _head_body(sp_ref, hp_ref, dinv_ref, b1_ref, bn1g_ref, bn1b_ref,
               batch_ref, fc1w_ref, fc1b_ref, bnfg_ref, bnfb_ref,
               fc2w_ref, fc2b_ref, out_ref):
    sagg = sp_ref[0, :N, :] + sp_ref[1, :N, :]
    h1 = (sagg + hp_ref[...]) * dinv_ref[...] + b1_ref[...]
    mean = jnp.mean(h1, axis=0, keepdims=True)
    var = jnp.mean((h1 - mean) ** 2, axis=0, keepdims=True)
    h1 = bn1g_ref[...] * (h1 - mean) * lax.rsqrt(var + 1e-5) + bn1b_ref[...]
    h1 = jnp.maximum(h1, 0.0)
    rows = lax.broadcasted_iota(jnp.int32, (G, N), 0)
    onehot = jnp.where(rows == batch_ref[...], 1.0, 0.0)
    sums = jnp.dot(onehot, h1, preferred_element_type=jnp.float32,
                   precision=lax.Precision.HIGHEST)
    cnt = jnp.sum(onehot, axis=1, keepdims=True)
    pooled = sums / jnp.maximum(cnt, 1.0)
    z = jnp.dot(pooled, fc1w_ref[...],
                preferred_element_type=jnp.float32) + fc1b_ref[...]
    m2 = jnp.mean(z, axis=0, keepdims=True)
    v2 = jnp.mean((z - m2) ** 2, axis=0, keepdims=True)
    z = bnfg_ref[...] * (z - m2) * lax.rsqrt(v2 + 1e-5) + bnfb_ref[...]
    z = jnp.maximum(z, 0.0)
    out_ref[...] = jnp.dot(z, fc2w_ref[...],
                           preferred_element_type=jnp.float32) + fc2b_ref[...]


def kernel(x, edge_index, batch, W1, b1, bn1_g, bn1_b, fc1_W, fc1_b,
           bnf_g, bnf_b, fc2_W, fc2_b):
    src = edge_index[0]
    dst = edge_index[1]
    pad = TILES * PERP - E
    srcp = jnp.concatenate([src, jnp.zeros((pad,), jnp.int32)])
    dstp = jnp.concatenate([dst, jnp.full((pad,), N, jnp.int32)])
    srcb = srcp.reshape(NCHUNK, CHW)
    dstb = dstp.reshape(NCHUNK, CHW)
    dsta = dstp.reshape(TILES, PERP)

    h = pl.pallas_call(
        _matmul_body,
        out_shape=jax.ShapeDtypeStruct((N, H), jnp.float32),
    )(x, W1)
    degp = _deg_kernel(dsta)

    hp, dinv = pl.pallas_call(
        _scale_body,
        out_shape=[jax.ShapeDtypeStruct((N, H), jnp.float32),
                   jax.ShapeDtypeStruct((N, 1), jnp.float32)],
    )(h, degp)

    zeros = jnp.zeros((NP, H), jnp.float32)
    sp = _scatter_kernel(hp, srcb, dstb, zeros)

    out = pl.pallas_call(
        _head_body,
        out_shape=jax.ShapeDtypeStruct((G, 1), jnp.float32),
    )(sp, hp, dinv, b1.reshape(1, H), bn1_g.reshape(1, H),
      bn1_b.reshape(1, H), batch.reshape(1, N), fc1_W,
      fc1_b.reshape(1, H), bnf_g.reshape(1, H), bnf_b.reshape(1, H),
      fc2_W, fc2_b.reshape(1, 1))
    return out[:, 0]


# speedup vs baseline: 1.5550x; 1.5550x over previous
import functools

import jax
import jax.numpy as jnp
from jax import lax
from jax.experimental import pallas as pl
from jax.experimental.pallas import tpu as pltpu
from jax.experimental.pallas import tpu_sc as plsc

N = 10000
E = 320000
G = 64
D_IN = 128
H = 32

NC = 2
NS = 16
TILES = NC * NS
CHW = 128
PER = E // TILES
CH = -(-PER // CHW)
PERP = CH * CHW
NP = N + 112
ROWS = NP // NS

_mesh = plsc.VectorSubcoreMesh(core_axis_name="c", subcore_axis_name="s")


@functools.partial(
    pl.kernel,
    mesh=_mesh,
    out_type=jax.ShapeDtypeStruct((TILES, NP), jnp.float32),
    scratch_types=[
        pltpu.VMEM((PERP,), jnp.int32),
        pltpu.VMEM((NP,), jnp.float32),
    ],
    compiler_params=pltpu.CompilerParams(needs_layout_passes=False,
                                         use_tc_tiling_on_sc=False),
)
def _deg_kernel(dst_hbm, out_hbm, idx_v, deg_v):
    c = lax.axis_index("c")
    s = lax.axis_index("s")
    w = s * NC + c
    pltpu.sync_copy(dst_hbm.at[w], idx_v)
    zero16 = jnp.zeros((16,), jnp.float32)
    one16 = jnp.ones((16,), jnp.float32)

    def zb(i, carry):
        deg_v[pl.ds(i * 16, 16)] = zero16
        return carry

    lax.fori_loop(0, NP // 16, zb, 0, unroll=8)

    def sb(i, carry):
        idx = idx_v[pl.ds(i * 16, 16)]
        plsc.addupdate_scatter(deg_v, [idx], one16)
        return carry

    lax.fori_loop(0, PERP // 16, sb, 0, unroll=8)
    pltpu.sync_copy(deg_v, out_hbm.at[w])


@functools.partial(
    pl.kernel,
    mesh=_mesh,
    out_type=jax.ShapeDtypeStruct((NC, NP, H), jnp.float32),
    scratch_types=[
        pltpu.VMEM((CH, CHW), jnp.int32),
        pltpu.VMEM((CH, CHW), jnp.int32),
        pltpu.VMEM((CHW, H), jnp.float32),
        pltpu.VMEM((CHW, H), jnp.float32),
        pltpu.VMEM((CHW, H), jnp.float32),
        pltpu.VMEM((CHW, H), jnp.float32),
        pltpu.VMEM_SHARED((NP, H), jnp.float32),
        pltpu.SemaphoreType.DMA,
        pltpu.SemaphoreType.DMA,
        pltpu.SemaphoreType.DMA,
        pltpu.SemaphoreType.DMA,
    ],
    compiler_params=pltpu.CompilerParams(needs_layout_passes=False,
                                         use_tc_tiling_on_sc=False),
)
def _scatter_kernel(hp_hbm, src_hbm, dst_hbm, zeros_hbm, out_hbm,
                    src_v, dst_v, buf_a, buf_b, buf_c, buf_d, s_sh,
                    sem_a, sem_b, sem_c, sem_d):
    c = lax.axis_index("c")
    s = lax.axis_index("s")
    w = s * NC + c
    pltpu.sync_copy(src_hbm.at[w], src_v)
    pltpu.sync_copy(dst_hbm.at[w], dst_v)
    pltpu.sync_copy(zeros_hbm.at[pl.ds(s * ROWS, ROWS)],
                    s_sh.at[pl.ds(s * ROWS, ROWS)])
    plsc.subcore_barrier()

    bufs = (buf_a, buf_b, buf_c, buf_d)
    sems = (sem_a, sem_b, sem_c, sem_d)
    DEPTH = 4
    for p in range(DEPTH - 1):
        pltpu.async_copy(hp_hbm.at[src_v.at[p]], bufs[p], sems[p])

    def body(j, carry):
        slot = j % DEPTH
        nxt = j + DEPTH - 1
        for p in range(DEPTH):
            @pl.when(jnp.logical_and(nxt < CH, slot == p))
            def _(p=p):
                pltpu.async_copy(hp_hbm.at[src_v.at[nxt]],
                                 bufs[(p + DEPTH - 1) % DEPTH],
                                 sems[(p + DEPTH - 1) % DEPTH])
        for p in range(DEPTH):
            @pl.when(slot == p)
            def _(p=p):
                pltpu.make_async_copy(hp_hbm.at[src_v.at[j]],
                                      bufs[p], sems[p]).wait()
                pltpu.sync_copy(bufs[p], s_sh.at[dst_v.at[j]], add=True)
        return carry

    lax.fori_loop(0, CH, body, 0)
    plsc.subcore_barrier()
    pltpu.sync_copy(s_sh.at[pl.ds(s * ROWS, ROWS)],
                    out_hbm.at[c, pl.ds(s * ROWS, ROWS)])


def _matmul_body(x_ref, w1_ref, h_ref):
    h_ref[...] = jnp.dot(x_ref[...], w1_ref[...],
                         preferred_element_type=jnp.float32)


def _scale_body(h_ref, degp_ref, hp_ref, dinv_ref):
    deg = jnp.sum(degp_ref[...], axis=0)[:N] + 1.0
    dinv = lax.rsqrt(deg)
    hp_ref[...] = h_ref[...] * dinv[:, None]
    dinv_ref[...] = dinv[:, None]


def _head_body(sp_ref, hp_ref, dinv_ref, b1_ref, bn1g_ref, bn1b_ref,
               batch_ref, fc1w_ref, fc1b_ref, bnfg_ref, bnfb_ref,
               fc2w_ref, fc2b_ref, out_ref):
    sagg = sp_ref[0, :N, :] + sp_ref[1, :N, :]
    h1 = (sagg + hp_ref[...]) * dinv_ref[...] + b1_ref[...]
    mean = jnp.mean(h1, axis=0, keepdims=True)
    var = jnp.mean((h1 - mean) ** 2, axis=0, keepdims=True)
    h1 = bn1g_ref[...] * (h1 - mean) * lax.rsqrt(var + 1e-5) + bn1b_ref[...]
    h1 = jnp.maximum(h1, 0.0)
    rows = lax.broadcasted_iota(jnp.int32, (G, N), 0)
    onehot = jnp.where(rows == batch_ref[...], 1.0, 0.0)
    sums = jnp.dot(onehot, h1, preferred_element_type=jnp.float32,
                   precision=lax.Precision.HIGHEST)
    cnt = jnp.sum(onehot, axis=1, keepdims=True)
    pooled = sums / jnp.maximum(cnt, 1.0)
    z = jnp.dot(pooled, fc1w_ref[...],
                preferred_element_type=jnp.float32) + fc1b_ref[...]
    m2 = jnp.mean(z, axis=0, keepdims=True)
    v2 = jnp.mean((z - m2) ** 2, axis=0, keepdims=True)
    z = bnfg_ref[...] * (z - m2) * lax.rsqrt(v2 + 1e-5) + bnfb_ref[...]
    z = jnp.maximum(z, 0.0)
    out_ref[...] = jnp.dot(z, fc2w_ref[...],
                           preferred_element_type=jnp.float32) + fc2b_ref[...]


def kernel(x, edge_index, batch, W1, b1, bn1_g, bn1_b, fc1_W, fc1_b,
           bnf_g, bnf_b, fc2_W, fc2_b):
    src = edge_index[0]
    dst = edge_index[1]
    pad = TILES * PERP - E
    srcp = jnp.concatenate([src, jnp.zeros((pad,), jnp.int32)])
    dstp = jnp.concatenate([dst, jnp.full((pad,), N, jnp.int32)])
    srcb = srcp.reshape(TILES, CH, CHW)
    dstb = dstp.reshape(TILES, CH, CHW)
    dsta = dstp.reshape(TILES, PERP)

    h = pl.pallas_call(
        _matmul_body,
        out_shape=jax.ShapeDtypeStruct((N, H), jnp.float32),
    )(x, W1)
    degp = _deg_kernel(dsta)

    hp, dinv = pl.pallas_call(
        _scale_body,
        out_shape=[jax.ShapeDtypeStruct((N, H), jnp.float32),
                   jax.ShapeDtypeStruct((N, 1), jnp.float32)],
    )(h, degp)

    zeros = jnp.zeros((NP, H), jnp.float32)
    sp = _scatter_kernel(hp, srcb, dstb, zeros)

    out = pl.pallas_call(
        _head_body,
        out_shape=jax.ShapeDtypeStruct((G, 1), jnp.float32),
    )(sp, hp, dinv, b1.reshape(1, H), bn1_g.reshape(1, H),
      bn1_b.reshape(1, H), batch.reshape(1, N), fc1_W,
      fc1_b.reshape(1, H), bnf_g.reshape(1, H), bnf_b.reshape(1, H),
      fc2_W, fc2_b.reshape(1, 1))
    return out[:, 0]


# speedup vs baseline: 1.5746x; 1.0126x over previous
import functools

import jax
import jax.numpy as jnp
from jax import lax
from jax.experimental import pallas as pl
from jax.experimental.pallas import tpu as pltpu
from jax.experimental.pallas import tpu_sc as plsc

N = 10000
E = 320000
G = 64
D_IN = 128
H = 32

NC = 2
NS = 16
TILES = NC * NS
CHW = 128
PER = E // TILES
CH = -(-PER // CHW)
PERP = CH * CHW
NP = N + 112
ROWS = NP // NS

_mesh = plsc.VectorSubcoreMesh(core_axis_name="c", subcore_axis_name="s")


@functools.partial(
    pl.kernel,
    mesh=_mesh,
    out_type=jax.ShapeDtypeStruct((TILES, NP), jnp.float32),
    scratch_types=[
        pltpu.VMEM((PERP,), jnp.int32),
        pltpu.VMEM((NP,), jnp.float32),
    ],
    compiler_params=pltpu.CompilerParams(needs_layout_passes=False,
                                         use_tc_tiling_on_sc=False),
)
def _deg_kernel(dst_hbm, out_hbm, idx_v, deg_v):
    c = lax.axis_index("c")
    s = lax.axis_index("s")
    w = s * NC + c
    pltpu.sync_copy(dst_hbm.at[w], idx_v)
    zero16 = jnp.zeros((16,), jnp.float32)
    one16 = jnp.ones((16,), jnp.float32)

    def zb(i, carry):
        deg_v[pl.ds(i * 16, 16)] = zero16
        return carry

    lax.fori_loop(0, NP // 16, zb, 0, unroll=8)

    def sb(i, carry):
        idx = idx_v[pl.ds(i * 16, 16)]
        plsc.addupdate_scatter(deg_v, [idx], one16)
        return carry

    lax.fori_loop(0, PERP // 16, sb, 0, unroll=8)
    pltpu.sync_copy(deg_v, out_hbm.at[w])


@functools.partial(
    pl.kernel,
    mesh=_mesh,
    out_type=jax.ShapeDtypeStruct((NC, NP, H), jnp.float32),
    scratch_types=[
        pltpu.VMEM((CH, CHW), jnp.int32),
        pltpu.VMEM((CH, CHW), jnp.int32),
        pltpu.VMEM((CHW, H), jnp.float32),
        pltpu.VMEM((CHW, H), jnp.float32),
        pltpu.VMEM((CHW, H), jnp.float32),
        pltpu.VMEM((CHW, H), jnp.float32),
        pltpu.VMEM_SHARED((NP, H), jnp.float32),
        pltpu.SemaphoreType.DMA,
        pltpu.SemaphoreType.DMA,
        pltpu.SemaphoreType.DMA,
        pltpu.SemaphoreType.DMA,
    ],
    compiler_params=pltpu.CompilerParams(needs_layout_passes=False,
                                         use_tc_tiling_on_sc=False),
)
def _scatter_kernel(hp_hbm, src_hbm, dst_hbm, out_hbm,
                    src_v, dst_v, buf_a, buf_b, buf_c, buf_d, s_sh,
                    sem_a, sem_b, sem_c, sem_d):
    c = lax.axis_index("c")
    s = lax.axis_index("s")
    w = s * NC + c
    pltpu.sync_copy(src_hbm.at[w], src_v)
    pltpu.sync_copy(dst_hbm.at[w], dst_v)
    zero16 = jnp.zeros((16,), jnp.float32)

    def zb(i, carry):
        buf_a[i // (H // 16), pl.ds((i % (H // 16)) * 16, 16)] = zero16
        return carry

    lax.fori_loop(0, CHW * H // 16, zb, 0, unroll=8)
    for p in range(4):
        pltpu.sync_copy(buf_a, s_sh.at[pl.ds(s * ROWS + p * CHW, CHW)])
    pltpu.sync_copy(buf_a.at[pl.ds(0, ROWS - 4 * CHW)],
                    s_sh.at[pl.ds(s * ROWS + 4 * CHW, ROWS - 4 * CHW)])
    plsc.subcore_barrier()

    bufs = (buf_a, buf_b, buf_c, buf_d)
    sems = (sem_a, sem_b, sem_c, sem_d)
    DEPTH = 4
    for p in range(DEPTH - 1):
        pltpu.async_copy(hp_hbm.at[src_v.at[p]], bufs[p], sems[p])

    def body(j, carry):
        slot = j % DEPTH
        nxt = j + DEPTH - 1
        for p in range(DEPTH):
            @pl.when(jnp.logical_and(nxt < CH, slot == p))
            def _(p=p):
                pltpu.async_copy(hp_hbm.at[src_v.at[nxt]],
                                 bufs[(p + DEPTH - 1) % DEPTH],
                                 sems[(p + DEPTH - 1) % DEPTH])
        for p in range(DEPTH):
            @pl.when(slot == p)
            def _(p=p):
                pltpu.make_async_copy(hp_hbm.at[src_v.at[j]],
                                      bufs[p], sems[p]).wait()
                pltpu.sync_copy(bufs[p], s_sh.at[dst_v.at[j]], add=True)
        return carry

    lax.fori_loop(0, CH, body, 0)
    plsc.subcore_barrier()
    pltpu.sync_copy(s_sh.at[pl.ds(s * ROWS, ROWS)],
                    out_hbm.at[c, pl.ds(s * ROWS, ROWS)])


def _matmul_body(x_ref, w1_ref, h_ref):
    h_ref[...] = jnp.dot(x_ref[...], w1_ref[...],
                         preferred_element_type=jnp.float32)


def _scale_body(h_ref, degp_ref, hp_ref, dinv_ref):
    deg = jnp.sum(degp_ref[...], axis=0)[:N] + 1.0
    dinv = lax.rsqrt(deg)
    hp_ref[...] = h_ref[...] * dinv[:, None]
    dinv_ref[...] = dinv[:, None]


def _head_body(sp_ref, hp_ref, dinv_ref, b1_ref, bn1g_ref, bn1b_ref,
               batch_ref, fc1w_ref, fc1b_ref, bnfg_ref, bnfb_ref,
               fc2w_ref, fc2b_ref, out_ref):
    sagg = sp_ref[0, :N, :] + sp_ref[1, :N, :]
    h1 = (sagg + hp_ref[...]) * dinv_ref[...] + b1_ref[...]
    mean = jnp.mean(h1, axis=0, keepdims=True)
    var = jnp.mean((h1 - mean) ** 2, axis=0, keepdims=True)
    h1 = bn1g_ref[...] * (h1 - mean) * lax.rsqrt(var + 1e-5) + bn1b_ref[...]
    h1 = jnp.maximum(h1, 0.0)
    rows = lax.broadcasted_iota(jnp.int32, (G, N), 0)
    onehot = jnp.where(rows == batch_ref[...], 1.0, 0.0)
    sums = jnp.dot(onehot, h1, preferred_element_type=jnp.float32,
                   precision=lax.Precision.HIGHEST)
    cnt = jnp.sum(onehot, axis=1, keepdims=True)
    pooled = sums / jnp.maximum(cnt, 1.0)
    z = jnp.dot(pooled, fc1w_ref[...],
                preferred_element_type=jnp.float32) + fc1b_ref[...]
    m2 = jnp.mean(z, axis=0, keepdims=True)
    v2 = jnp.mean((z - m2) ** 2, axis=0, keepdims=True)
    z = bnfg_ref[...] * (z - m2) * lax.rsqrt(v2 + 1e-5) + bnfb_ref[...]
    z = jnp.maximum(z, 0.0)
    out_ref[...] = jnp.dot(z, fc2w_ref[...],
                           preferred_element_type=jnp.float32) + fc2b_ref[...]


def kernel(x, edge_index, batch, W1, b1, bn1_g, bn1_b, fc1_W, fc1_b,
           bnf_g, bnf_b, fc2_W, fc2_b):
    src = edge_index[0]
    dst = edge_index[1]
    pad = TILES * PERP - E
    srcp = jnp.concatenate([src, jnp.zeros((pad,), jnp.int32)])
    dstp = jnp.concatenate([dst, jnp.full((pad,), N, jnp.int32)])
    srcb = srcp.reshape(TILES, CH, CHW)
    dstb = dstp.reshape(TILES, CH, CHW)
    dsta = dstp.reshape(TILES, PERP)

    h = pl.pallas_call(
        _matmul_body,
        out_shape=jax.ShapeDtypeStruct((N, H), jnp.float32),
    )(x, W1)
    degp = _deg_kernel(dsta)

    hp, dinv = pl.pallas_call(
        _scale_body,
        out_shape=[jax.ShapeDtypeStruct((N, H), jnp.float32),
                   jax.ShapeDtypeStruct((N, 1), jnp.float32)],
    )(h, degp)

    sp = _scatter_kernel(hp, srcb, dstb)

    out = pl.pallas_call(
        _head_body,
        out_shape=jax.ShapeDtypeStruct((G, 1), jnp.float32),
    )(sp, hp, dinv, b1.reshape(1, H), bn1_g.reshape(1, H),
      bn1_b.reshape(1, H), batch.reshape(1, N), fc1_W,
      fc1_b.reshape(1, H), bnf_g.reshape(1, H), bnf_b.reshape(1, H),
      fc2_W, fc2_b.reshape(1, 1))
    return out[:, 0]


# speedup vs baseline: 1.5985x; 1.0151x over previous
import functools

import jax
import jax.numpy as jnp
from jax import lax
from jax.experimental import pallas as pl
from jax.experimental.pallas import tpu as pltpu
from jax.experimental.pallas import tpu_sc as plsc

N = 10000
E = 320000
G = 64
D_IN = 128
H = 32

NC = 2
NS = 16
TILES = NC * NS
CHW = 128
PER = E // TILES
CH = -(-PER // CHW)
PERP = CH * CHW
NP = N + 112
ROWS = NP // NS

_mesh = plsc.VectorSubcoreMesh(core_axis_name="c", subcore_axis_name="s")


@functools.partial(
    pl.kernel,
    mesh=_mesh,
    out_type=jax.ShapeDtypeStruct((TILES, NP), jnp.float32),
    scratch_types=[
        pltpu.VMEM((PERP,), jnp.int32),
        pltpu.VMEM((NP,), jnp.float32),
    ],
    compiler_params=pltpu.CompilerParams(needs_layout_passes=False,
                                         use_tc_tiling_on_sc=False),
)
def _deg_kernel(dst_hbm, out_hbm, idx_v, deg_v):
    c = lax.axis_index("c")
    s = lax.axis_index("s")
    w = s * NC + c
    pltpu.sync_copy(dst_hbm.at[w], idx_v)
    zero16 = jnp.zeros((16,), jnp.float32)
    one16 = jnp.ones((16,), jnp.float32)

    def zb(i, carry):
        deg_v[pl.ds(i * 16, 16)] = zero16
        return carry

    lax.fori_loop(0, NP // 16, zb, 0, unroll=8)

    def sb(i, carry):
        idx = idx_v[pl.ds(i * 16, 16)]
        plsc.addupdate_scatter(deg_v, [idx], one16)
        return carry

    lax.fori_loop(0, PERP // 16, sb, 0, unroll=8)
    pltpu.sync_copy(deg_v, out_hbm.at[w])


@functools.partial(
    pl.kernel,
    mesh=_mesh,
    out_type=jax.ShapeDtypeStruct((NC, NP, H), jnp.float32),
    scratch_types=[
        pltpu.VMEM((CH, CHW), jnp.int32),
        pltpu.VMEM((CH, CHW), jnp.int32),
        pltpu.VMEM((CHW, H), jnp.float32),
        pltpu.VMEM((CHW, H), jnp.float32),
        pltpu.VMEM((CHW, H), jnp.float32),
        pltpu.VMEM((CHW, H), jnp.float32),
        pltpu.VMEM((CHW, H), jnp.float32),
        pltpu.VMEM((CHW, H), jnp.float32),
        pltpu.VMEM((CHW, H), jnp.float32),
        pltpu.VMEM((CHW, H), jnp.float32),
        pltpu.VMEM_SHARED((NP, H), jnp.float32),
        pltpu.SemaphoreType.DMA,
        pltpu.SemaphoreType.DMA,
        pltpu.SemaphoreType.DMA,
        pltpu.SemaphoreType.DMA,
        pltpu.SemaphoreType.DMA,
        pltpu.SemaphoreType.DMA,
        pltpu.SemaphoreType.DMA,
        pltpu.SemaphoreType.DMA,
    ],
    compiler_params=pltpu.CompilerParams(needs_layout_passes=False,
                                         use_tc_tiling_on_sc=False),
)
def _scatter_kernel(hp_hbm, src_hbm, dst_hbm, out_hbm,
                    src_v, dst_v, buf_a, buf_b, buf_c, buf_d,
                    buf_e, buf_f, buf_g, buf_h, s_sh,
                    sem_a, sem_b, sem_c, sem_d,
                    sem_e, sem_f, sem_g, sem_h):
    c = lax.axis_index("c")
    s = lax.axis_index("s")
    w = s * NC + c
    pltpu.sync_copy(src_hbm.at[w], src_v)
    pltpu.sync_copy(dst_hbm.at[w], dst_v)
    zero16 = jnp.zeros((16,), jnp.float32)

    def zb(i, carry):
        buf_a[i // (H // 16), pl.ds((i % (H // 16)) * 16, 16)] = zero16
        return carry

    lax.fori_loop(0, CHW * H // 16, zb, 0, unroll=8)
    for p in range(4):
        pltpu.sync_copy(buf_a, s_sh.at[pl.ds(s * ROWS + p * CHW, CHW)])
    pltpu.sync_copy(buf_a.at[pl.ds(0, ROWS - 4 * CHW)],
                    s_sh.at[pl.ds(s * ROWS + 4 * CHW, ROWS - 4 * CHW)])
    plsc.subcore_barrier()

    bufs = (buf_a, buf_b, buf_c, buf_d, buf_e, buf_f, buf_g, buf_h)
    sems = (sem_a, sem_b, sem_c, sem_d, sem_e, sem_f, sem_g, sem_h)
    DEPTH = 8
    for p in range(DEPTH - 1):
        pltpu.async_copy(hp_hbm.at[src_v.at[p]], bufs[p], sems[p])

    def body(j, carry):
        slot = j % DEPTH
        nxt = j + DEPTH - 1
        for p in range(DEPTH):
            @pl.when(jnp.logical_and(nxt < CH, slot == p))
            def _(p=p):
                pltpu.async_copy(hp_hbm.at[src_v.at[nxt]],
                                 bufs[(p + DEPTH - 1) % DEPTH],
                                 sems[(p + DEPTH - 1) % DEPTH])
        for p in range(DEPTH):
            @pl.when(slot == p)
            def _(p=p):
                pltpu.make_async_copy(hp_hbm.at[src_v.at[j]],
                                      bufs[p], sems[p]).wait()
                pltpu.sync_copy(bufs[p], s_sh.at[dst_v.at[j]], add=True)
        return carry

    lax.fori_loop(0, CH, body, 0)
    plsc.subcore_barrier()
    pltpu.sync_copy(s_sh.at[pl.ds(s * ROWS, ROWS)],
                    out_hbm.at[c, pl.ds(s * ROWS, ROWS)])


def _matmul_body(x_ref, w1_ref, h_ref):
    h_ref[...] = jnp.dot(x_ref[...], w1_ref[...],
                         preferred_element_type=jnp.float32)


def _scale_body(h_ref, degp_ref, hp_ref, dinv_ref):
    deg = jnp.sum(degp_ref[...], axis=0)[:N] + 1.0
    dinv = lax.rsqrt(deg)
    hp_ref[...] = h_ref[...] * dinv[:, None]
    dinv_ref[...] = dinv[:, None]


def _head_body(sp_ref, hp_ref, dinv_ref, b1_ref, bn1g_ref, bn1b_ref,
               batch_ref, fc1w_ref, fc1b_ref, bnfg_ref, bnfb_ref,
               fc2w_ref, fc2b_ref, out_ref):
    sagg = sp_ref[0, :N, :] + sp_ref[1, :N, :]
    h1 = (sagg + hp_ref[...]) * dinv_ref[...] + b1_ref[...]
    mean = jnp.mean(h1, axis=0, keepdims=True)
    var = jnp.mean((h1 - mean) ** 2, axis=0, keepdims=True)
    h1 = bn1g_ref[...] * (h1 - mean) * lax.rsqrt(var + 1e-5) + bn1b_ref[...]
    h1 = jnp.maximum(h1, 0.0)
    rows = lax.broadcasted_iota(jnp.int32, (G, N), 0)
    onehot = jnp.where(rows == batch_ref[...], 1.0, 0.0)
    sums = jnp.dot(onehot, h1, preferred_element_type=jnp.float32,
                   precision=lax.Precision.HIGHEST)
    cnt = jnp.sum(onehot, axis=1, keepdims=True)
    pooled = sums / jnp.maximum(cnt, 1.0)
    z = jnp.dot(pooled, fc1w_ref[...],
                preferred_element_type=jnp.float32) + fc1b_ref[...]
    m2 = jnp.mean(z, axis=0, keepdims=True)
    v2 = jnp.mean((z - m2) ** 2, axis=0, keepdims=True)
    z = bnfg_ref[...] * (z - m2) * lax.rsqrt(v2 + 1e-5) + bnfb_ref[...]
    z = jnp.maximum(z, 0.0)
    out_ref[...] = jnp.dot(z, fc2w_ref[...],
                           preferred_element_type=jnp.float32) + fc2b_ref[...]


def kernel(x, edge_index, batch, W1, b1, bn1_g, bn1_b, fc1_W, fc1_b,
           bnf_g, bnf_b, fc2_W, fc2_b):
    src = edge_index[0]
    dst = edge_index[1]
    pad = TILES * PERP - E
    srcp = jnp.concatenate([src, jnp.zeros((pad,), jnp.int32)])
    dstp = jnp.concatenate([dst, jnp.full((pad,), N, jnp.int32)])
    srcb = srcp.reshape(TILES, CH, CHW)
    dstb = dstp.reshape(TILES, CH, CHW)
    dsta = dstp.reshape(TILES, PERP)

    h = pl.pallas_call(
        _matmul_body,
        out_shape=jax.ShapeDtypeStruct((N, H), jnp.float32),
    )(x, W1)
    degp = _deg_kernel(dsta)

    hp, dinv = pl.pallas_call(
        _scale_body,
        out_shape=[jax.ShapeDtypeStruct((N, H), jnp.float32),
                   jax.ShapeDtypeStruct((N, 1), jnp.float32)],
    )(h, degp)

    sp = _scatter_kernel(hp, srcb, dstb)

    out = pl.pallas_call(
        _head_body,
        out_shape=jax.ShapeDtypeStruct((G, 1), jnp.float32),
    )(sp, hp, dinv, b1.reshape(1, H), bn1_g.reshape(1, H),
      bn1_b.reshape(1, H), batch.reshape(1, N), fc1_W,
      fc1_b.reshape(1, H), bnf_g.reshape(1, H), bnf_b.reshape(1, H),
      fc2_W, fc2_b.reshape(1, 1))
    return out[:, 0]
